# baseline (speedup 1.0000x reference)
import functools

import jax
import jax.numpy as jnp
from jax.experimental import pallas as pl

NUM_EXPERTS = 8
TOP_K = 2
D_IN = 768
D_OUT = 768
N_TOK = 8192
LB_WEIGHT = 0.01

BLOCK_N = 512


def _moe_block_kernel(x_ref, wr_ref, br_ref, we_ref, be_ref,
                      out_ref, prob_ref, aux_ref):
    i = pl.program_id(0)
    nblocks = pl.num_programs(0)

    x = x_ref[:]
    logits = jax.lax.dot_general(
        x, wr_ref[:], (((1,), (1,)), ((), ())),
        preferred_element_type=jnp.float32) + br_ref[:]
    m = jnp.max(logits, axis=-1, keepdims=True)
    ex = jnp.exp(logits - m)
    probs = ex / jnp.sum(ex, axis=-1, keepdims=True)

    @pl.when(i == 0)
    def _init():
        prob_ref[:] = jnp.zeros_like(prob_ref)

    prob_ref[:] += jnp.sum(probs, axis=0, keepdims=True)

    eids = jax.lax.broadcasted_iota(jnp.int32, probs.shape, 1)
    i1 = jnp.argmax(probs, axis=-1)
    w1 = jnp.max(probs, axis=-1)
    masked = jnp.where(eids == i1[:, None], -jnp.inf, probs)
    i2 = jnp.argmax(masked, axis=-1)
    w2 = jnp.max(masked, axis=-1)
    s = w1 + w2
    c1 = (w1 / s)[:, None]
    c2 = (w2 / s)[:, None]
    combine = (jnp.where(eids == i1[:, None], c1, 0.0)
               + jnp.where(eids == i2[:, None], c2, 0.0))

    acc = jax.lax.dot_general(
        combine, be_ref[:], (((1,), (0,)), ((), ())),
        preferred_element_type=jnp.float32)
    xb = x.astype(jnp.bfloat16)
    for e in range(NUM_EXPERTS):
        y_e = jax.lax.dot_general(
            xb, we_ref[e], (((1,), (1,)), ((), ())),
            preferred_element_type=jnp.float32)
        acc += combine[:, e:e + 1] * y_e
    out_ref[:] = acc

    @pl.when(i == nblocks - 1)
    def _finalize():
        p = prob_ref[:] / N_TOK
        d = p - (1.0 / NUM_EXPERTS)
        aux_ref[:] = jnp.reshape(jnp.mean(d * d) * LB_WEIGHT, (1, 1))


@functools.partial(jax.jit, static_argnames=())
def _moe(x, Wr, br2, We, be):
    nblocks = N_TOK // BLOCK_N
    out, _prob, aux = pl.pallas_call(
        _moe_block_kernel,
        grid=(nblocks,),
        in_specs=[
            pl.BlockSpec((BLOCK_N, D_IN), lambda i: (i, 0)),
            pl.BlockSpec((NUM_EXPERTS, D_IN), lambda i: (0, 0)),
            pl.BlockSpec((1, NUM_EXPERTS), lambda i: (0, 0)),
            pl.BlockSpec((NUM_EXPERTS, D_OUT, D_IN), lambda i: (0, 0, 0)),
            pl.BlockSpec((NUM_EXPERTS, D_OUT), lambda i: (0, 0)),
        ],
        out_specs=[
            pl.BlockSpec((BLOCK_N, D_OUT), lambda i: (i, 0)),
            pl.BlockSpec((1, NUM_EXPERTS), lambda i: (0, 0)),
            pl.BlockSpec((1, 1), lambda i: (0, 0)),
        ],
        out_shape=[
            jax.ShapeDtypeStruct((N_TOK, D_OUT), jnp.float32),
            jax.ShapeDtypeStruct((1, NUM_EXPERTS), jnp.float32),
            jax.ShapeDtypeStruct((1, 1), jnp.float32),
        ],
    )(x, Wr, br2, We, be)
    return out, aux[0, 0]


def kernel(x, Wr, br, We, be):
    return _moe(x, Wr, br.reshape(1, NUM_EXPERTS), We.astype(jnp.bfloat16), be)


# speedup vs baseline: 1.1129x; 1.1129x over previous
import functools

import jax
import jax.numpy as jnp
from jax.experimental import pallas as pl

NUM_EXPERTS = 8
TOP_K = 2
D_IN = 768
D_OUT = 768
N_TOK = 8192
LB_WEIGHT = 0.01

BLOCK_N = 1024


def _moe_block_kernel(x_ref, wr_ref, br_ref, we_ref, be_ref,
                      out_ref, prob_ref, aux_ref):
    i = pl.program_id(0)
    nblocks = pl.num_programs(0)

    x = x_ref[:]
    logits = jax.lax.dot_general(
        x, wr_ref[:], (((1,), (1,)), ((), ())),
        preferred_element_type=jnp.float32) + br_ref[:]
    m = jnp.max(logits, axis=-1, keepdims=True)
    ex = jnp.exp(logits - m)
    probs = ex / jnp.sum(ex, axis=-1, keepdims=True)

    @pl.when(i == 0)
    def _init():
        prob_ref[:] = jnp.zeros_like(prob_ref)

    prob_ref[:] += jnp.sum(probs, axis=0, keepdims=True)

    eids = jax.lax.broadcasted_iota(jnp.int32, probs.shape, 1)
    i1 = jnp.argmax(probs, axis=-1)
    w1 = jnp.max(probs, axis=-1)
    masked = jnp.where(eids == i1[:, None], -jnp.inf, probs)
    i2 = jnp.argmax(masked, axis=-1)
    w2 = jnp.max(masked, axis=-1)
    s = w1 + w2
    c1 = (w1 / s)[:, None]
    c2 = (w2 / s)[:, None]
    combine = (jnp.where(eids == i1[:, None], c1, 0.0)
               + jnp.where(eids == i2[:, None], c2, 0.0))

    acc = jax.lax.dot_general(
        combine, be_ref[:], (((1,), (0,)), ((), ())),
        preferred_element_type=jnp.float32)
    for e in range(NUM_EXPERTS):
        y_e = jax.lax.dot_general(
            x, we_ref[e], (((1,), (1,)), ((), ())),
            preferred_element_type=jnp.float32)
        acc += combine[:, e:e + 1] * y_e
    out_ref[:] = acc

    @pl.when(i == nblocks - 1)
    def _finalize():
        p = prob_ref[:] / N_TOK
        d = p - (1.0 / NUM_EXPERTS)
        aux_ref[:] = jnp.reshape(jnp.mean(d * d) * LB_WEIGHT, (1, 1))


@functools.partial(jax.jit, static_argnames=())
def _moe(x, Wr, br2, We, be):
    nblocks = N_TOK // BLOCK_N
    out, _prob, aux = pl.pallas_call(
        _moe_block_kernel,
        grid=(nblocks,),
        in_specs=[
            pl.BlockSpec((BLOCK_N, D_IN), lambda i: (i, 0)),
            pl.BlockSpec((NUM_EXPERTS, D_IN), lambda i: (0, 0)),
            pl.BlockSpec((1, NUM_EXPERTS), lambda i: (0, 0)),
            pl.BlockSpec((NUM_EXPERTS, D_OUT, D_IN), lambda i: (0, 0, 0)),
            pl.BlockSpec((NUM_EXPERTS, D_OUT), lambda i: (0, 0)),
        ],
        out_specs=[
            pl.BlockSpec((BLOCK_N, D_OUT), lambda i: (i, 0)),
            pl.BlockSpec((1, NUM_EXPERTS), lambda i: (0, 0)),
            pl.BlockSpec((1, 1), lambda i: (0, 0)),
        ],
        out_shape=[
            jax.ShapeDtypeStruct((N_TOK, D_OUT), jnp.float32),
            jax.ShapeDtypeStruct((1, NUM_EXPERTS), jnp.float32),
            jax.ShapeDtypeStruct((1, 1), jnp.float32),
        ],
    )(x, Wr, br2, We, be)
    return out, aux[0, 0]


def kernel(x, Wr, br, We, be):
    return _moe(x, Wr, br.reshape(1, NUM_EXPERTS), We, be)
